# baseline (speedup 1.0000x reference)
import jax
import jax.numpy as jnp
from jax import lax
from jax.experimental import pallas as pl
from jax.experimental.pallas import tpu as pltpu

_ROWS = 16384
_COLS = 2048
_CH = 1024
_NCH = _ROWS // _CH
_NBUF = 3
_NOUTER = 4
_INNER = 4


def _ring_kernel(x_hbm, m_hbm, o_hbm,
                 i0, i1, i2, o0, o1, o2, m_vmem,
                 si0, si1, si2, so0, so1, so2, sm):
    ibufs = (i0, i1, i2)
    obufs = (o0, o1, o2)
    sin = (si0, si1, si2)
    sout = (so0, so1, so2)

    pltpu.async_copy(m_hbm, m_vmem, sm).wait()

    for b in range(_NBUF):
        pltpu.async_copy(x_hbm.at[pl.ds(b * _CH, _CH), :], ibufs[b], sin[b])

    m = None
    for j in range(_NCH):
        b = j % _NBUF
        r0 = j * _CH
        pltpu.make_async_copy(x_hbm.at[pl.ds(r0, _CH), :], ibufs[b],
                              sin[b]).wait()
        if m is None:
            m = m_vmem[...]
        if j >= _NBUF:
            p0 = (j - _NBUF) * _CH
            pltpu.make_async_copy(obufs[b], o_hbm.at[pl.ds(p0, _CH), :],
                                  sout[b]).wait()
        x = ibufs[b][...]
        obufs[b][...] = jnp.where(m != 0.0, jnp.tanh(x), x)
        pltpu.async_copy(obufs[b], o_hbm.at[pl.ds(r0, _CH), :], sout[b])
        n = j + _NBUF
        if n < _NCH:
            pltpu.async_copy(x_hbm.at[pl.ds(n * _CH, _CH), :], ibufs[b],
                             sin[b])

    for b in range(_NBUF):
        r0 = (_NCH - _NBUF + b) * _CH
        pltpu.make_async_copy(obufs[(_NCH - _NBUF + b) % _NBUF],
                              o_hbm.at[pl.ds(r0, _CH), :],
                              sout[(_NCH - _NBUF + b) % _NBUF]).wait()


def kernel(x, mask):
    m = mask.astype(jnp.float32).reshape(1, _COLS)
    return pl.pallas_call(
        _ring_kernel,
        in_specs=[
            pl.BlockSpec(memory_space=pltpu.MemorySpace.HBM),
            pl.BlockSpec(memory_space=pltpu.MemorySpace.HBM),
        ],
        out_specs=pl.BlockSpec(memory_space=pltpu.MemorySpace.HBM),
        out_shape=jax.ShapeDtypeStruct((_ROWS, _COLS), jnp.float32),
        scratch_shapes=(
            [pltpu.VMEM((_CH, _COLS), jnp.float32)] * (2 * _NBUF)
            + [pltpu.VMEM((1, _COLS), jnp.float32)]
            + [pltpu.SemaphoreType.DMA] * (2 * _NBUF + 1)
        ),
        compiler_params=pltpu.CompilerParams(
            vmem_limit_bytes=128 * 1024 * 1024,
        ),
    )(x, m)


# speedup vs baseline: 1.0044x; 1.0044x over previous
import jax
import jax.numpy as jnp
from jax.experimental import pallas as pl

_ROWS = 16384
_COLS = 2048
_BLOCK_ROWS = 1024


def _masked_tanh_kernel(x_ref, m_ref, o_ref):
    x = x_ref[...]
    m = m_ref[...]
    o_ref[...] = jnp.where(m != 0.0, jnp.tanh(x), x)


def kernel(x, mask):
    m = mask.astype(jnp.float32).reshape(1, _COLS)
    return pl.pallas_call(
        _masked_tanh_kernel,
        grid=(_ROWS // _BLOCK_ROWS,),
        in_specs=[
            pl.BlockSpec((_BLOCK_ROWS, _COLS), lambda i: (i, 0)),
            pl.BlockSpec((1, _COLS), lambda i: (0, 0)),
        ],
        out_specs=pl.BlockSpec((_BLOCK_ROWS, _COLS), lambda i: (i, 0)),
        out_shape=jax.ShapeDtypeStruct((_ROWS, _COLS), jnp.float32),
    )(x, m)
